# baseline (speedup 1.0000x reference)
import jax
import jax.numpy as jnp
from jax import lax
from jax.experimental import pallas as pl
from jax.experimental.pallas import tpu as pltpu
from jax.experimental.pallas import tpu_sc as plsc

B = 1024
HIST = 200
D = 32
NSAMP = 64
NC, NS = 2, 16
NW = NC * NS
BW = B // NW
LP = 208
NBLK = LP // 16
S1 = 104
S2 = HIST - S1


def _sc_body(dt_hbm, tl_hbm, ll_hbm, nz_hbm, word_hbm, doc_hbm, lab_hbm,
             outdoc_hbm, labout_hbm, nzout_hbm,
             dt_v, tl_v, ll_v, nz_v, w_v, d_v, lab_v, nzrow_v, out_v, wts_v,
             sem1, sem2, semd):
    wid = lax.axis_index("c") * NS + lax.axis_index("s")
    base = wid * BW

    pltpu.sync_copy(dt_hbm.at[pl.ds(base, BW)], dt_v)
    pltpu.sync_copy(tl_hbm.at[pl.ds(base, BW)], tl_v)
    pltpu.sync_copy(ll_hbm.at[pl.ds(base, BW)], ll_v)

    pltpu.async_copy(doc_hbm.at[tl_v], d_v, semd).wait()
    pltpu.async_copy(lab_hbm.at[ll_v], lab_v, semd).wait()
    pltpu.sync_copy(lab_v, labout_hbm.at[pl.ds(base, BW)])

    @pl.when(wid == 0)
    def _():
        pltpu.sync_copy(nz_hbm, nz_v)
        pltpu.async_copy(doc_hbm.at[nz_v], nzrow_v, semd).wait()
        pltpu.sync_copy(nzrow_v, nzout_hbm)

    zero16 = jnp.zeros((16,), jnp.float32)
    for r in range(HIST, LP):
        w_v[r, pl.ds(0, 16)] = zero16
        w_v[r, pl.ds(16, 16)] = zero16

    lane = lax.broadcasted_iota(jnp.int32, (16,), 0)
    rows_c = [blk * 16 + lane for blk in range(NBLK)]
    tail_mask = lane < (HIST - 16 * (NBLK - 1))

    def b_body(b, carry):
        cp1 = pltpu.async_copy(word_hbm.at[dt_v.at[b, pl.ds(0, S1)]],
                               w_v.at[pl.ds(0, S1)], sem1)
        cp2 = pltpu.async_copy(word_hbm.at[dt_v.at[b, pl.ds(S1, S2)]],
                               w_v.at[pl.ds(S1, S2)], sem2)
        cp1.wait()
        cp2.wait()

        bfull = jnp.full((16,), b, jnp.int32)

        def j_body(j, sc):
            colj = jnp.full((16,), j, jnp.int32)
            dj = plsc.load_gather(d_v, [bfull, colj])
            return tuple(sc[k] + plsc.load_gather(w_v, [rows_c[k], colj]) * dj
                         for k in range(NBLK))

        sc0 = tuple(jnp.zeros((16,), jnp.float32) for _ in range(NBLK))
        sc = list(lax.fori_loop(0, D, j_body, sc0))
        sc[NBLK - 1] = jnp.where(tail_mask, sc[NBLK - 1], -1e30)

        m = sc[0]
        for k in range(1, NBLK):
            m = jnp.maximum(m, sc[k])
        mm = jnp.max(m)
        es = [jnp.exp(s - mm) for s in sc]
        tot = es[0]
        for k in range(1, NBLK):
            tot = tot + es[k]
        inv = 1.0 / jnp.full((16,), jnp.sum(tot), jnp.float32)
        for k in range(NBLK):
            wts_v[pl.ds(k * 16, 16)] = es[k] * inv

        def l_body(i, acc):
            a0, a1 = acc
            for u in range(4):
                l = i * 4 + u
                wt = plsc.load_gather(wts_v, [jnp.full((16,), l, jnp.int32)])
                a0 = a0 + wt * w_v[l, pl.ds(0, 16)]
                a1 = a1 + wt * w_v[l, pl.ds(16, 16)]
            return (a0, a1)

        a0, a1 = lax.fori_loop(0, HIST // 4, l_body, (zero16, zero16))
        out_v[b, pl.ds(0, 16)] = a0
        out_v[b, pl.ds(16, 16)] = a1
        return carry

    lax.fori_loop(0, BW, b_body, 0)
    pltpu.sync_copy(out_v, outdoc_hbm.at[pl.ds(base, BW)])


def _sc_call(dt, tl, ll, noise_ids, word_table, doc_table, label_table):
    mesh = plsc.VectorSubcoreMesh(core_axis_name="c", subcore_axis_name="s",
                                  num_cores=NC, num_subcores=NS)
    out_types = (jax.ShapeDtypeStruct((B, D), jnp.float32),
                 jax.ShapeDtypeStruct((B, D), jnp.float32),
                 jax.ShapeDtypeStruct((NSAMP, D), jnp.float32))
    scratch = [
        pltpu.VMEM((BW, HIST), jnp.int32),
        pltpu.VMEM((BW,), jnp.int32),
        pltpu.VMEM((BW,), jnp.int32),
        pltpu.VMEM((NSAMP,), jnp.int32),
        pltpu.VMEM((LP, D), jnp.float32),
        pltpu.VMEM((BW, D), jnp.float32),
        pltpu.VMEM((BW, D), jnp.float32),
        pltpu.VMEM((NSAMP, D), jnp.float32),
        pltpu.VMEM((BW, D), jnp.float32),
        pltpu.VMEM((LP,), jnp.float32),
        pltpu.SemaphoreType.DMA,
        pltpu.SemaphoreType.DMA,
        pltpu.SemaphoreType.DMA,
    ]
    k = pl.kernel(_sc_body, out_type=out_types, mesh=mesh,
                  scratch_types=scratch,
                  compiler_params=pltpu.CompilerParams(
                      use_tc_tiling_on_sc=False,
                      needs_layout_passes=False))
    return k(dt, tl, ll, noise_ids, word_table, doc_table, label_table)


def _score_body(y_ref, x0_ref, lab_ref, out_ref):
    y = y_ref[...]
    lab = lab_ref[...]
    x0 = x0_ref[...]
    dn = (((1,), (1,)), ((), ()))
    s = lax.dot_general(y, lab, dn, preferred_element_type=jnp.float32)
    s0 = lax.dot_general(x0, lab, dn, preferred_element_type=jnp.float32)
    out_ref[...] = jnp.maximum(s - s0 + 1.0, 0.0)


def kernel(dt, tl, ll, num_sampled, opt, noise_ids, word_table, doc_table,
           label_table):
    del num_sampled, opt
    outdoc, labr, nzrows = _sc_call(dt, tl, ll, noise_ids, word_table,
                                    doc_table, label_table)
    y = jnp.concatenate([outdoc[1:], nzrows], axis=0)
    x0 = outdoc[0:1]
    return pl.pallas_call(
        _score_body,
        out_shape=jax.ShapeDtypeStruct((B + NSAMP - 1, B), jnp.float32),
    )(y, x0, labr)
